# baseline (speedup 1.0000x reference)
import functools

import jax
import jax.numpy as jnp
from jax import lax
from jax.experimental import pallas as pl
from jax.experimental.pallas import tpu as pltpu
from jax.experimental.pallas import tpu_sc as plsc

_VOCAB = 100000
_SEQ = 200
_DIM = 64
_BATCH = 4096

_NC = 2
_NS = 16
_NW = _NC * _NS
_TOTAL = _BATCH * _SEQ
_CH = 128
_B_TC = 0
_N_TC = _B_TC * _SEQ
_SC_TOTAL = _TOTAL - _N_TC
_PER_W = _SC_TOTAL // _NW
_NCHUNK = _PER_W // _CH
_NBUF = 8
_FD = 5
_PD = 3


def _sc_embed(idx_flat, token_table, pos_table):
    mesh = plsc.VectorSubcoreMesh(core_axis_name="c", subcore_axis_name="s")

    @functools.partial(
        pl.kernel,
        out_type=jax.ShapeDtypeStruct((_TOTAL, _DIM), jnp.float32),
        mesh=mesh,
        compiler_params=pltpu.CompilerParams(use_tc_tiling_on_sc=False),
        scratch_types=[
            pltpu.VMEM_SHARED((2 * _SEQ, _DIM), jnp.float32),
            pltpu.VMEM((_PER_W,), jnp.int32),
            [pltpu.VMEM((_CH, _DIM), jnp.float32)] * _NBUF,
            [pltpu.SemaphoreType.DMA] * _NBUF,
            [pltpu.SemaphoreType.DMA] * _NBUF,
            [pltpu.SemaphoreType.DMA] * _NBUF,
        ],
    )
    def k(idx_hbm, tok_hbm, pos_hbm, out_hbm,
          pos2_s, idx_v, rows, fsem, gsem, osem):
        wid = lax.axis_index("s") * _NC + lax.axis_index("c")
        base = _N_TC + wid * _PER_W
        pltpu.sync_copy(idx_hbm.at[pl.ds(base, _PER_W)], idx_v)

        @pl.when(lax.axis_index("s") == 0)
        def _():
            pltpu.sync_copy(pos_hbm, pos2_s.at[pl.ds(0, _SEQ)])
            pltpu.sync_copy(pos_hbm, pos2_s.at[pl.ds(_SEQ, _SEQ)])

        plsc.subcore_barrier()

        def fill(j, b):
            pj = lax.rem(j * _CH, _SEQ)
            pltpu.async_copy(pos2_s.at[pl.ds(pj, _CH)], rows[b], fsem[b])

        def fill_wait(b):
            pltpu.make_async_copy(
                pos2_s.at[pl.ds(0, _CH)], rows[b], fsem[b]).wait()

        def gather_add(j, b):
            pltpu.async_copy(
                tok_hbm.at[idx_v.at[pl.ds(j * _CH, _CH)]], rows[b], gsem[b],
                add=True)

        def gather_wait(b):
            pltpu.make_async_copy(
                tok_hbm.at[idx_v.at[pl.ds(0, _CH)]], rows[b], gsem[b]).wait()

        def store(j, b):
            return pltpu.make_async_copy(
                rows[b], out_hbm.at[pl.ds(base + j * _CH, _CH)], osem[b])

        for b in range(_FD):
            fill(b, b)
        for b in range(_PD):
            fill_wait(b)
            gather_add(b, b)

        @pl.loop(0, _NCHUNK, step=_NBUF)
        def _chunks(i0):
            for b in range(_NBUF):
                i = i0 + b
                jf = i + _FD
                bf = (b + _FD) % _NBUF

                @pl.when(jf < _NCHUNK)
                def _():
                    @pl.when(jf >= _NBUF)
                    def _():
                        store(0, bf).wait()

                    fill(jf, bf)

                jg = i + _PD
                bg = (b + _PD) % _NBUF

                @pl.when(jg < _NCHUNK)
                def _():
                    fill_wait(bg)
                    gather_add(jg, bg)

                gather_wait(b)
                store(i, b).start()

        for b in range(_NBUF):
            store(0, b).wait()

    return k(idx_flat, token_table, pos_table)


def kernel(inputs, token_table, pos_table):
    idx_flat = jnp.reshape(inputs, (-1,)).astype(jnp.int32)
    out = _sc_embed(idx_flat, token_table, pos_table)
    if _B_TC:
        tc_part = (jnp.take(token_table,
                            inputs[:_B_TC].astype(jnp.int32), axis=0)
                   + pos_table[None, :, :]).reshape(_N_TC, _DIM)
        out = lax.dynamic_update_slice(out, tc_part, (0, 0))
    out = out.reshape(_BATCH, _SEQ, _DIM)
    zero = lax.optimization_barrier(jnp.zeros((), jnp.float32))
    return out + zero


# speedup vs baseline: 1.3685x; 1.3685x over previous
import functools

import jax
import jax.numpy as jnp
from jax import lax
from jax.experimental import pallas as pl
from jax.experimental.pallas import tpu as pltpu
from jax.experimental.pallas import tpu_sc as plsc

_VOCAB = 100000
_SEQ = 200
_DIM = 64
_BATCH = 4096

_NC = 2
_NS = 16
_NW = _NC * _NS
_TOTAL = _BATCH * _SEQ
_CH = 128
_B_TC = 0
_N_TC = _B_TC * _SEQ
_SC_TOTAL = _TOTAL - _N_TC
_PER_W = _SC_TOTAL // _NW
_NCHUNK = _PER_W // _CH
_NBUF = 8
_FD = 5
_PD = 3


def _sc_embed(idx_flat, token_table, pos_table):
    mesh = plsc.VectorSubcoreMesh(core_axis_name="c", subcore_axis_name="s")

    @functools.partial(
        pl.kernel,
        out_type=jax.ShapeDtypeStruct((_TOTAL, _DIM), jnp.float32),
        mesh=mesh,
        compiler_params=pltpu.CompilerParams(use_tc_tiling_on_sc=False),
        scratch_types=[
            pltpu.VMEM_SHARED((2 * _SEQ, _DIM), jnp.float32),
            pltpu.VMEM((_PER_W,), jnp.int32),
            [pltpu.VMEM((_CH, _DIM), jnp.float32)] * _NBUF,
            [pltpu.SemaphoreType.DMA] * _NBUF,
            [pltpu.SemaphoreType.DMA] * _NBUF,
            [pltpu.SemaphoreType.DMA] * _NBUF,
        ],
    )
    def k(idx_hbm, tok_hbm, pos_hbm, out_hbm,
          pos2_s, idx_v, rows, fsem, gsem, osem):
        wid = lax.axis_index("s") * _NC + lax.axis_index("c")
        base = _N_TC + wid * _PER_W
        pltpu.sync_copy(idx_hbm.at[pl.ds(base, _PER_W)], idx_v)

        @pl.when(lax.axis_index("s") == 0)
        def _():
            pltpu.sync_copy(pos_hbm, pos2_s.at[pl.ds(0, _SEQ)])
            pltpu.sync_copy(pos_hbm, pos2_s.at[pl.ds(_SEQ, _SEQ)])

        plsc.subcore_barrier()

        def fill(j, b):
            pj = lax.rem(j * _CH, _SEQ)
            pltpu.async_copy(pos2_s.at[pl.ds(pj, _CH)], rows[b], fsem[b])

        def fill_wait(b):
            pltpu.make_async_copy(
                pos2_s.at[pl.ds(0, _CH)], rows[b], fsem[b]).wait()

        def gather_add(j, b):
            pltpu.async_copy(
                tok_hbm.at[idx_v.at[pl.ds(j * _CH, _CH)]], rows[b], gsem[b],
                add=True)

        def gather_wait(b):
            pltpu.make_async_copy(
                tok_hbm.at[idx_v.at[pl.ds(0, _CH)]], rows[b], gsem[b]).wait()

        def store(j, b):
            return pltpu.make_async_copy(
                rows[b], out_hbm.at[pl.ds(base + j * _CH, _CH)], osem[b])

        for b in range(_FD):
            fill(b, b)
        for b in range(_PD):
            fill_wait(b)
            gather_add(b, b)

        @pl.loop(0, _NCHUNK, step=_NBUF)
        def _chunks(i0):
            for b in range(_NBUF):
                i = i0 + b
                jf = i + _FD
                bf = (b + _FD) % _NBUF

                @pl.when(jf < _NCHUNK)
                def _():
                    @pl.when(jf >= _NBUF)
                    def _():
                        store(0, bf).wait()

                    fill(jf, bf)

                jg = i + _PD
                bg = (b + _PD) % _NBUF

                @pl.when(jg < _NCHUNK)
                def _():
                    fill_wait(bg)
                    gather_add(jg, bg)

                gather_wait(b)
                store(i, b).start()

        for b in range(_NBUF):
            store(0, b).wait()

    return k(idx_flat, token_table, pos_table)


def kernel(inputs, token_table, pos_table):
    idx_flat = jnp.reshape(inputs, (-1,)).astype(jnp.int32)
    out = _sc_embed(idx_flat, token_table, pos_table)
    if _B_TC:
        tc_part = (jnp.take(token_table,
                            inputs[:_B_TC].astype(jnp.int32), axis=0)
                   + pos_table[None, :, :]).reshape(_N_TC, _DIM)
        out = lax.dynamic_update_slice(out, tc_part, (0, 0))
    return out.reshape(_BATCH, _SEQ, _DIM)
